# baseline (speedup 1.0000x reference)
import functools

import jax
import jax.numpy as jnp
from jax import lax
from jax.experimental import pallas as pl
from jax.experimental.pallas import tpu as pltpu
from jax.experimental.pallas import tpu_sc as plsc

N = 512
N_IN = 128
N_OUT = 64
NW = 32
RPW = N // NW
NH = N // 2
HB = N // 4
JB = N // 2


def _abpk_body(xe_ref, xo_ref, w_ref, b_ref, apk_ref, bpk_ref):
    xe = xe_ref[...]
    xo = xo_ref[...]
    W1 = w_ref[:, :N_IN]
    W2 = w_ref[:, N_IN:]
    bh = 0.5 * b_ref[...]
    dn = (((1,), (1,)), ((), ()))
    ae = lax.dot_general(xe, W1, dn, preferred_element_type=jnp.float32) + bh
    ao = lax.dot_general(xo, W1, dn, preferred_element_type=jnp.float32) + bh
    be = lax.dot_general(xe, W2, dn, preferred_element_type=jnp.float32) + bh
    bo = lax.dot_general(xo, W2, dn, preferred_element_type=jnp.float32) + bh
    apk_ref[...] = jnp.concatenate([ae, ao], axis=1)
    bpk_ref[...] = jnp.concatenate([be, bo], axis=1)


_mesh = plsc.VectorSubcoreMesh(core_axis_name="c", subcore_axis_name="s")


@functools.partial(
    pl.kernel,
    mesh=_mesh,
    out_type=jax.ShapeDtypeStruct((N, N, N_OUT), jnp.float32),
    scratch_types=[
        pltpu.VMEM((NH, 2 * N_OUT), jnp.float32),
        pltpu.VMEM((NH, 2 * N_OUT), jnp.float32),
        pltpu.VMEM((JB, N_OUT), jnp.float32),
        pltpu.VMEM((JB, N_OUT), jnp.float32),
        pltpu.SemaphoreType.DMA,
        pltpu.SemaphoreType.DMA,
    ],
)
def _sc_fill(apk_hbm, bpk_hbm, out_hbm, a_vm, b_vm, buf0, buf1, sem0, sem1):
    wid = lax.axis_index("s") * 2 + lax.axis_index("c")
    pltpu.sync_copy(apk_hbm, a_vm)
    pltpu.sync_copy(bpk_hbm, b_vm)
    zero = jnp.zeros((16,), jnp.float32)
    bufs = (buf0, buf1)
    sems = (sem0, sem1)
    row0 = wid * RPW

    @pl.loop(0, RPW, step=2)
    def _rowpair(base):
        @pl.when(base > 0)
        def _():
            for s in range(2):
                pltpu.make_async_copy(
                    bufs[s], out_hbm.at[row0, pl.ds(s * JB, JB)], sems[s]
                ).wait()

        qb = (row0 + base) // 2
        for b in range(2):
            i = row0 + base + b
            off = b * N_OUT
            a_i = [a_vm[qb, pl.ds(off + c * 16, 16)] for c in range(4)]
            b_i = [b_vm[qb, pl.ds(off + c * 16, 16)] for c in range(4)]
            for h in range(2):
                lo, hi = h * HB, h * HB + HB
                buf = bufs[h]
                if b == 1:
                    pltpu.make_async_copy(
                        buf, out_hbm.at[row0, pl.ds(h * JB, JB)], sems[h]
                    ).wait()
                lowhi = jnp.minimum(jnp.maximum(qb, lo), hi)
                uplo = jnp.minimum(jnp.maximum(qb + 1, lo), hi)

                @plsc.parallel_loop(lo, lowhi, unroll=8)
                def _lower(q):
                    for c in range(4):
                        buf[2 * q - 2 * lo, pl.ds(c * 16, 16)] = (
                            a_vm[q, pl.ds(c * 16, 16)] + b_i[c])
                        buf[2 * q + 1 - 2 * lo, pl.ds(c * 16, 16)] = (
                            a_vm[q, pl.ds(N_OUT + c * 16, 16)] + b_i[c])

                @plsc.parallel_loop(uplo, hi, unroll=8)
                def _upper(q):
                    for c in range(4):
                        buf[2 * q - 2 * lo, pl.ds(c * 16, 16)] = (
                            a_i[c] + b_vm[q, pl.ds(c * 16, 16)])
                        buf[2 * q + 1 - 2 * lo, pl.ds(c * 16, 16)] = (
                            a_i[c] + b_vm[q, pl.ds(N_OUT + c * 16, 16)])

                @pl.when((qb >= lo) & (qb < hi))
                def _fix():
                    loc = 2 * qb - 2 * lo
                    if b == 0:
                        for c in range(4):
                            buf[loc, pl.ds(c * 16, 16)] = zero
                            buf[loc + 1, pl.ds(c * 16, 16)] = (
                                a_i[c] + b_vm[qb, pl.ds(N_OUT + c * 16, 16)])
                    else:
                        for c in range(4):
                            buf[loc, pl.ds(c * 16, 16)] = (
                                a_vm[qb, pl.ds(c * 16, 16)] + b_i[c])
                            buf[loc + 1, pl.ds(c * 16, 16)] = zero

                pltpu.async_copy(buf, out_hbm.at[i, pl.ds(h * JB, JB)], sems[h])

    for s in range(2):
        pltpu.make_async_copy(
            bufs[s], out_hbm.at[row0, pl.ds(s * JB, JB)], sems[s]
        ).wait()


def kernel(inputs, W, b):
    x = inputs
    b2 = b.reshape(1, N_OUT)
    A_pk, B_pk = pl.pallas_call(
        _abpk_body,
        out_shape=[
            jax.ShapeDtypeStruct((NH, 2 * N_OUT), jnp.float32),
            jax.ShapeDtypeStruct((NH, 2 * N_OUT), jnp.float32),
        ],
    )(x[0::2], x[1::2], W, b2)
    return _sc_fill(A_pk, B_pk)


# speedup vs baseline: 1.0291x; 1.0291x over previous
import functools

import jax
import jax.numpy as jnp
from jax import lax
from jax.experimental import pallas as pl
from jax.experimental.pallas import tpu as pltpu
from jax.experimental.pallas import tpu_sc as plsc

N = 512
N_IN = 128
N_OUT = 64
NW = 32
RPW = N // NW
NH = N // 2
HB = N // 4
JB = N // 2


def _abpk_body(xe_ref, xo_ref, w_ref, b_ref, apk_ref, bpk_ref):
    xe = xe_ref[...]
    xo = xo_ref[...]
    W1 = w_ref[:, :N_IN]
    W2 = w_ref[:, N_IN:]
    bh = 0.5 * b_ref[...]
    dn = (((1,), (1,)), ((), ()))
    ae = lax.dot_general(xe, W1, dn, preferred_element_type=jnp.float32) + bh
    ao = lax.dot_general(xo, W1, dn, preferred_element_type=jnp.float32) + bh
    be = lax.dot_general(xe, W2, dn, preferred_element_type=jnp.float32) + bh
    bo = lax.dot_general(xo, W2, dn, preferred_element_type=jnp.float32) + bh
    apk_ref[...] = jnp.concatenate([ae, ao], axis=1)
    bpk_ref[...] = jnp.concatenate([be, bo], axis=1)


_mesh = plsc.VectorSubcoreMesh(core_axis_name="c", subcore_axis_name="s")


@functools.partial(
    pl.kernel,
    mesh=_mesh,
    out_type=jax.ShapeDtypeStruct((N, N, N_OUT), jnp.float32),
    scratch_types=[
        pltpu.VMEM((NH, 2 * N_OUT), jnp.float32),
        pltpu.VMEM((NH, 2 * N_OUT), jnp.float32),
        pltpu.VMEM((JB, N_OUT), jnp.float32),
        pltpu.VMEM((JB, N_OUT), jnp.float32),
        pltpu.SemaphoreType.DMA,
        pltpu.SemaphoreType.DMA,
    ],
)
def _sc_fill(apk_hbm, bpk_hbm, out_hbm, a_vm, b_vm, buf0, buf1, sem0, sem1):
    wid = lax.axis_index("s") * 2 + lax.axis_index("c")
    pltpu.sync_copy(apk_hbm, a_vm)
    pltpu.sync_copy(bpk_hbm, b_vm)
    zero = jnp.zeros((16,), jnp.float32)
    bufs = (buf0, buf1)
    sems = (sem0, sem1)
    row0 = wid * RPW

    @pl.loop(0, RPW, step=2)
    def _rowpair(base):
        qb = (row0 + base) // 2
        for b in range(2):
            i = row0 + base + b
            off = b * N_OUT
            a_i = [a_vm[qb, pl.ds(off + c * 16, 16)] for c in range(4)]
            b_i = [b_vm[qb, pl.ds(off + c * 16, 16)] for c in range(4)]
            for h in range(2):
                lo, hi = h * HB, h * HB + HB
                buf = bufs[h]
                if b == 1:
                    pltpu.make_async_copy(
                        buf, out_hbm.at[row0, pl.ds(h * JB, JB)], sems[h]
                    ).wait()
                else:
                    @pl.when(base > 0)
                    def _():
                        pltpu.make_async_copy(
                            buf, out_hbm.at[row0, pl.ds(h * JB, JB)], sems[h]
                        ).wait()
                lowhi = jnp.minimum(jnp.maximum(qb, lo), hi)
                uplo = jnp.minimum(jnp.maximum(qb + 1, lo), hi)

                @plsc.parallel_loop(lo, lowhi, unroll=4)
                def _lower(q):
                    for c in range(4):
                        buf[2 * q - 2 * lo, pl.ds(c * 16, 16)] = (
                            a_vm[q, pl.ds(c * 16, 16)] + b_i[c])
                        buf[2 * q + 1 - 2 * lo, pl.ds(c * 16, 16)] = (
                            a_vm[q, pl.ds(N_OUT + c * 16, 16)] + b_i[c])

                @plsc.parallel_loop(uplo, hi, unroll=4)
                def _upper(q):
                    for c in range(4):
                        buf[2 * q - 2 * lo, pl.ds(c * 16, 16)] = (
                            a_i[c] + b_vm[q, pl.ds(c * 16, 16)])
                        buf[2 * q + 1 - 2 * lo, pl.ds(c * 16, 16)] = (
                            a_i[c] + b_vm[q, pl.ds(N_OUT + c * 16, 16)])

                @pl.when((qb >= lo) & (qb < hi))
                def _fix():
                    loc = 2 * qb - 2 * lo
                    if b == 0:
                        for c in range(4):
                            buf[loc, pl.ds(c * 16, 16)] = zero
                            buf[loc + 1, pl.ds(c * 16, 16)] = (
                                a_i[c] + b_vm[qb, pl.ds(N_OUT + c * 16, 16)])
                    else:
                        for c in range(4):
                            buf[loc, pl.ds(c * 16, 16)] = (
                                a_vm[qb, pl.ds(c * 16, 16)] + b_i[c])
                            buf[loc + 1, pl.ds(c * 16, 16)] = zero

                pltpu.async_copy(buf, out_hbm.at[i, pl.ds(h * JB, JB)], sems[h])

    for s in range(2):
        pltpu.make_async_copy(
            bufs[s], out_hbm.at[row0, pl.ds(s * JB, JB)], sems[s]
        ).wait()


def kernel(inputs, W, b):
    x = inputs
    b2 = b.reshape(1, N_OUT)
    A_pk, B_pk = pl.pallas_call(
        _abpk_body,
        out_shape=[
            jax.ShapeDtypeStruct((NH, 2 * N_OUT), jnp.float32),
            jax.ShapeDtypeStruct((NH, 2 * N_OUT), jnp.float32),
        ],
    )(x[0::2], x[1::2], W, b2)
    return _sc_fill(A_pk, B_pk)
